# Initial kernel scaffold; baseline (speedup 1.0000x reference)
import functools

import jax
import jax.numpy as jnp
from jax import lax
from jax.experimental import pallas as pl
from jax.experimental.pallas import tpu as pltpu
from jax.experimental.pallas import tpu_sc as plsc

N_NODES = 10000
D = 128
ALPHA = 0.1

NC = 2
NS = 16
NW = NC * NS
CHUNK = 128
LANES = 16

ROWS_PER_TILE = N_NODES // NS
ZCHUNK = 125


def _sc_spmm_body(x_hbm, col_hbm, row_hbm, ea_hbm, out_hbm,
                  idx_v, row_v, ea_v, rows_v, acc, sem):
    c = lax.axis_index("c")
    s = lax.axis_index("s")

    def zero_body(r, _):
        for j in range(D // LANES):
            rows_v[r, pl.ds(j * LANES, LANES)] = jnp.zeros((LANES,), jnp.float32)
        return 0
    lax.fori_loop(0, ZCHUNK, zero_body, 0)
    for k in range(ROWS_PER_TILE // ZCHUNK):
        pltpu.sync_copy(rows_v.at[pl.ds(0, ZCHUNK)],
                        acc.at[pl.ds(s * ROWS_PER_TILE + k * ZCHUNK, ZCHUNK)])
    plsc.subcore_barrier()

    epw = col_hbm.shape[0] // NW
    nchunks = epw // CHUNK
    base = (c * NS + s) * epw

    def chunk_body(k, _):
        off = base + k * CHUNK
        pltpu.sync_copy(col_hbm.at[pl.ds(off, CHUNK)], idx_v)
        pltpu.sync_copy(row_hbm.at[pl.ds(off, CHUNK)], row_v)
        pltpu.sync_copy(ea_hbm.at[pl.ds(off, CHUNK)], ea_v)
        pltpu.async_copy(x_hbm.at[idx_v], rows_v, sem).wait()

        def row_body(r, _):
            a = ea_v[r]
            for j in range(D // LANES):
                rows_v[r, pl.ds(j * LANES, LANES)] = (
                    rows_v[r, pl.ds(j * LANES, LANES)] * a)
            return 0
        lax.fori_loop(0, CHUNK, row_body, 0)

        pltpu.sync_copy(rows_v, acc.at[row_v], add=True)
        return 0
    lax.fori_loop(0, nchunks, chunk_body, 0)

    plsc.subcore_barrier()
    for k in range(ROWS_PER_TILE // ZCHUNK):
        r0 = s * ROWS_PER_TILE + k * ZCHUNK
        pltpu.sync_copy(acc.at[pl.ds(r0, ZCHUNK)],
                        out_hbm.at[c, pl.ds(r0, ZCHUNK)])


@jax.jit
def _sc_spmm(x, col, row, ea):
    mesh = plsc.VectorSubcoreMesh(core_axis_name="c", subcore_axis_name="s")
    f = pl.kernel(
        _sc_spmm_body,
        out_type=jax.ShapeDtypeStruct((NC, N_NODES, D), jnp.float32),
        mesh=mesh,
        scratch_types=[
            pltpu.VMEM((CHUNK,), jnp.int32),
            pltpu.VMEM((CHUNK,), jnp.int32),
            pltpu.VMEM((CHUNK,), jnp.float32),
            pltpu.VMEM((CHUNK, D), jnp.float32),
            pltpu.VMEM_SHARED((N_NODES, D), jnp.float32),
            pltpu.SemaphoreType.DMA,
        ],
    )
    return f(x, col, row, ea)


def _tc_body(p_ref, ix_ref, w_ref, o_ref):
    hidden = (1.0 - ALPHA) * (p_ref[0] + p_ref[1]) + ALPHA * ix_ref[...]
    o_ref[...] = jnp.dot(hidden, w_ref[...], preferred_element_type=jnp.float32)


@jax.jit
def _tc_combine_matmul(partials, init_x, weight):
    blk = 1000
    grid = (N_NODES // blk,)
    return pl.pallas_call(
        _tc_body,
        grid=grid,
        in_specs=[
            pl.BlockSpec((NC, blk, D), lambda i: (0, i, 0)),
            pl.BlockSpec((blk, D), lambda i: (i, 0)),
            pl.BlockSpec((D, D), lambda i: (0, 0)),
        ],
        out_specs=pl.BlockSpec((blk, D), lambda i: (i, 0)),
        out_shape=jax.ShapeDtypeStruct((N_NODES, D), jnp.float32),
    )(partials, init_x, weight)


def kernel(x, edge_index, edge_attr, init_x, weight):
    e = edge_index.shape[1]
    epw = ((e + NW * CHUNK - 1) // (NW * CHUNK)) * CHUNK
    ep = epw * NW
    row = jnp.asarray(edge_index[0], jnp.int32)
    col = jnp.asarray(edge_index[1], jnp.int32)
    pad = ep - e
    row = jnp.pad(row, (0, pad))
    col = jnp.pad(col, (0, pad))
    ea = jnp.pad(jnp.asarray(edge_attr, jnp.float32), (0, pad))
    partials = _sc_spmm(x, col, row, ea)
    return _tc_combine_matmul(partials, init_x, weight)


# speedup vs baseline: 4.2036x; 4.2036x over previous
import functools

import jax
import jax.numpy as jnp
from jax import lax
from jax.experimental import pallas as pl
from jax.experimental.pallas import tpu as pltpu
from jax.experimental.pallas import tpu_sc as plsc

N_NODES = 10000
D = 128
ALPHA = 0.1

NC = 2
NS = 16
NW = NC * NS
CHUNK = 128
LANES = 16

N_PAD = 10240
ROWS_PER_TILE = N_PAD // NS
ZCHUNK = 128


def _sc_spmm_body(x_hbm, col_hbm, row_hbm, ea_hbm, out_hbm,
                  idx_v, row_v, ea_v, rows_v, acc, sem):
    c = lax.axis_index("c")
    s = lax.axis_index("s")

    def zero_body(r, _):
        for j in range(D // LANES):
            rows_v[r, pl.ds(j * LANES, LANES)] = jnp.zeros((LANES,), jnp.float32)
        return 0
    lax.fori_loop(0, ZCHUNK, zero_body, 0)
    for k in range(ROWS_PER_TILE // ZCHUNK):
        pltpu.sync_copy(rows_v.at[pl.ds(0, ZCHUNK)],
                        acc.at[pl.ds(s * ROWS_PER_TILE + k * ZCHUNK, ZCHUNK)])
    plsc.subcore_barrier()

    epw = col_hbm.shape[0] // NW
    nchunks = epw // CHUNK
    base = (c * NS + s) * epw

    def chunk_body(k, _):
        off = base + k * CHUNK
        pltpu.sync_copy(col_hbm.at[pl.ds(off, CHUNK)], idx_v)
        pltpu.sync_copy(row_hbm.at[pl.ds(off, CHUNK)], row_v)
        pltpu.sync_copy(ea_hbm.at[pl.ds(off, CHUNK)], ea_v)
        pltpu.async_copy(x_hbm.at[idx_v], rows_v, sem).wait()

        def group_body(g, _):
            av = ea_v[pl.ds(g * LANES, LANES)]
            for i in range(LANES):
                a = av[i]
                r = g * LANES + i
                for j in range(D // LANES):
                    rows_v[r, pl.ds(j * LANES, LANES)] = (
                        rows_v[r, pl.ds(j * LANES, LANES)] * a)
            return 0
        lax.fori_loop(0, CHUNK // LANES, group_body, 0)

        pltpu.sync_copy(rows_v, acc.at[row_v], add=True)
        return 0
    lax.fori_loop(0, nchunks, chunk_body, 0)

    plsc.subcore_barrier()
    for k in range(ROWS_PER_TILE // ZCHUNK):
        r0 = s * ROWS_PER_TILE + k * ZCHUNK
        pltpu.sync_copy(acc.at[pl.ds(r0, ZCHUNK)],
                        out_hbm.at[c, pl.ds(r0, ZCHUNK)])


@jax.jit
def _sc_spmm(x, col, row, ea):
    mesh = plsc.VectorSubcoreMesh(core_axis_name="c", subcore_axis_name="s")
    f = pl.kernel(
        _sc_spmm_body,
        out_type=jax.ShapeDtypeStruct((NC, N_PAD, D), jnp.float32),
        mesh=mesh,
        scratch_types=[
            pltpu.VMEM((CHUNK,), jnp.int32),
            pltpu.VMEM((CHUNK,), jnp.int32),
            pltpu.VMEM((CHUNK,), jnp.float32),
            pltpu.VMEM((CHUNK, D), jnp.float32),
            pltpu.VMEM_SHARED((N_PAD, D), jnp.float32),
            pltpu.SemaphoreType.DMA,
        ],
    )
    return f(x, col, row, ea)


def _tc_body(p_ref, ix_ref, w_ref, o_ref):
    hidden = (1.0 - ALPHA) * (p_ref[0] + p_ref[1]) + ALPHA * ix_ref[...]
    o_ref[...] = jnp.dot(hidden, w_ref[...], preferred_element_type=jnp.float32)


@jax.jit
def _tc_combine_matmul(partials, init_x, weight):
    blk = 1000
    grid = (N_NODES // blk,)
    return pl.pallas_call(
        _tc_body,
        grid=grid,
        in_specs=[
            pl.BlockSpec((NC, blk, D), lambda i: (0, i, 0)),
            pl.BlockSpec((blk, D), lambda i: (i, 0)),
            pl.BlockSpec((D, D), lambda i: (0, 0)),
        ],
        out_specs=pl.BlockSpec((blk, D), lambda i: (i, 0)),
        out_shape=jax.ShapeDtypeStruct((N_NODES, D), jnp.float32),
    )(partials, init_x, weight)


def kernel(x, edge_index, edge_attr, init_x, weight):
    e = edge_index.shape[1]
    epw = ((e + NW * CHUNK - 1) // (NW * CHUNK)) * CHUNK
    ep = epw * NW
    row = jnp.asarray(edge_index[0], jnp.int32)
    col = jnp.asarray(edge_index[1], jnp.int32)
    pad = ep - e
    row = jnp.pad(row, (0, pad))
    col = jnp.pad(col, (0, pad))
    ea = jnp.pad(jnp.asarray(edge_attr, jnp.float32), (0, pad))
    partials = _sc_spmm(x, col, row, ea)
    return _tc_combine_matmul(partials, init_x, weight)


# speedup vs baseline: 4.6370x; 1.1031x over previous
import jax
import jax.numpy as jnp
from jax import lax
from jax.experimental import pallas as pl
from jax.experimental.pallas import tpu as pltpu
from jax.experimental.pallas import tpu_sc as plsc

N_NODES = 10000
D = 128
ALPHA = 0.1

NC = 2
NS = 16
NW = NC * NS
CHUNK = 128
LANES = 16

N_PAD = 10240
ROWS_PER_TILE = N_PAD // NS
ZCHUNK = 128


def _sc_spmm_body(x_hbm, col_hbm, ea_hbm, row_hbm, out_hbm,
                  colc0, colc1, eac0, eac1, rowc0, rowc1, buf0, buf1, acc,
                  sem_g0, sem_g1):
    c = lax.axis_index("c")
    s = lax.axis_index("s")
    w = c * NS + s
    epw = col_hbm.shape[0] // NW
    nchunks = epw // CHUNK
    base = w * epw

    colc = (colc0, colc1)
    eac = (eac0, eac1)
    rowc = (rowc0, rowc1)
    bufs = (buf0, buf1)
    sem_g = (sem_g0, sem_g1)

    def load_idx(k, b):
        off = base + k * CHUNK
        pltpu.sync_copy(col_hbm.at[pl.ds(off, CHUNK)], colc[b])
        pltpu.sync_copy(ea_hbm.at[pl.ds(off, CHUNK)], eac[b])
        pltpu.sync_copy(row_hbm.at[pl.ds(off, CHUNK)], rowc[b])

    def issue_gather(b):
        pltpu.async_copy(x_hbm.at[colc[b]], bufs[b], sem_g[b])

    def wait_gather(b):
        pltpu.make_async_copy(x_hbm.at[colc[b]], bufs[b], sem_g[b]).wait()

    def zero_body(r, _):
        for j in range(D // LANES):
            buf0[r, pl.ds(j * LANES, LANES)] = jnp.zeros((LANES,), jnp.float32)
        return 0
    lax.fori_loop(0, ZCHUNK, zero_body, 0)
    for k in range(ROWS_PER_TILE // ZCHUNK):
        pltpu.sync_copy(buf0, acc.at[pl.ds(s * ROWS_PER_TILE + k * ZCHUNK, ZCHUNK)])
    plsc.subcore_barrier()

    load_idx(0, 0)
    issue_gather(0)
    wait_gather(0)

    def scale(b):
        buf = bufs[b]

        def group_body(g, _):
            av = eac[b][pl.ds(g * LANES, LANES)]
            for i in range(LANES):
                a = av[i]
                r = g * LANES + i
                for j in range(D // LANES):
                    buf[r, pl.ds(j * LANES, LANES)] = (
                        buf[r, pl.ds(j * LANES, LANES)] * a)
            return 0
        lax.fori_loop(0, CHUNK // LANES, group_body, 0)

    def scatter(b):
        pltpu.sync_copy(bufs[b], acc.at[rowc[b]], add=True)

    def process(k, b):
        load_idx(k + 1, 1 - b)
        issue_gather(1 - b)
        scale(b)
        wait_gather(1 - b)
        scatter(b)

    def pair_body(p, _):
        process(2 * p, 0)
        process(2 * p + 1, 1)
        return 0
    lax.fori_loop(0, (nchunks - 1) // 2, pair_body, 0)
    if nchunks % 2 == 0:
        process(nchunks - 2, 0)
        scale(1)
        scatter(1)
    else:
        scale(0)
        scatter(0)

    plsc.subcore_barrier()
    for k in range(ROWS_PER_TILE // ZCHUNK):
        r0 = s * ROWS_PER_TILE + k * ZCHUNK
        pltpu.sync_copy(acc.at[pl.ds(r0, ZCHUNK)],
                        out_hbm.at[c, pl.ds(r0, ZCHUNK)])


@jax.jit
def _sc_spmm(x, col, ea, row):
    mesh = plsc.VectorSubcoreMesh(core_axis_name="c", subcore_axis_name="s")
    f = pl.kernel(
        _sc_spmm_body,
        out_type=jax.ShapeDtypeStruct((NC, N_PAD, D), jnp.float32),
        mesh=mesh,
        scratch_types=[
            pltpu.VMEM((CHUNK,), jnp.int32),
            pltpu.VMEM((CHUNK,), jnp.int32),
            pltpu.VMEM((CHUNK,), jnp.float32),
            pltpu.VMEM((CHUNK,), jnp.float32),
            pltpu.VMEM((CHUNK,), jnp.int32),
            pltpu.VMEM((CHUNK,), jnp.int32),
            pltpu.VMEM((CHUNK, D), jnp.float32),
            pltpu.VMEM((CHUNK, D), jnp.float32),
            pltpu.VMEM_SHARED((N_PAD, D), jnp.float32),
            pltpu.SemaphoreType.DMA,
            pltpu.SemaphoreType.DMA,
        ],
    )
    return f(x, col, ea, row)


def _tc_body(p_ref, ix_ref, w_ref, o_ref):
    hidden = (1.0 - ALPHA) * (p_ref[0] + p_ref[1]) + ALPHA * ix_ref[...]
    o_ref[...] = jnp.dot(hidden, w_ref[...], preferred_element_type=jnp.float32)


@jax.jit
def _tc_combine_matmul(partials, init_x, weight):
    blk = 1000
    grid = (N_NODES // blk,)
    return pl.pallas_call(
        _tc_body,
        grid=grid,
        in_specs=[
            pl.BlockSpec((NC, blk, D), lambda i: (0, i, 0)),
            pl.BlockSpec((blk, D), lambda i: (i, 0)),
            pl.BlockSpec((D, D), lambda i: (0, 0)),
        ],
        out_specs=pl.BlockSpec((blk, D), lambda i: (i, 0)),
        out_shape=jax.ShapeDtypeStruct((N_NODES, D), jnp.float32),
    )(partials, init_x, weight)


def kernel(x, edge_index, edge_attr, init_x, weight):
    e = edge_index.shape[1]
    epw = ((e + NW * CHUNK - 1) // (NW * CHUNK)) * CHUNK
    ep = epw * NW
    pad = ep - e
    row = jnp.pad(jnp.asarray(edge_index[0], jnp.int32), (0, pad))
    col = jnp.pad(jnp.asarray(edge_index[1], jnp.int32), (0, pad))
    ea = jnp.pad(jnp.asarray(edge_attr, jnp.float32), (0, pad))
    partials = _sc_spmm(x, col, ea, row)
    return _tc_combine_matmul(partials, init_x, weight)


# speedup vs baseline: 5.3757x; 1.1593x over previous
import jax
import jax.numpy as jnp
from jax import lax
from jax.experimental import pallas as pl
from jax.experimental.pallas import tpu as pltpu
from jax.experimental.pallas import tpu_sc as plsc

N_NODES = 10000
D = 128
ALPHA = 0.1

NC = 2
NS = 16
NW = NC * NS
CHUNK = 128
LANES = 16

N_PAD = 10240
ROWS_PER_TILE = N_PAD // NS
ZCHUNK = 128


def _sc_spmm_body(x_hbm, idx_hbm, out_hbm,
                  idx0, idx1, buf0, buf1, acc,
                  sem_i0, sem_i1, sem_g0, sem_g1):
    c = lax.axis_index("c")
    s = lax.axis_index("s")
    w = c * NS + s
    nchunks = idx_hbm.shape[1]

    idxv = (idx0, idx1)
    bufs = (buf0, buf1)
    sem_i = (sem_i0, sem_i1)
    sem_g = (sem_g0, sem_g1)

    def issue_idx(k, b):
        pltpu.async_copy(idx_hbm.at[w, k], idxv[b], sem_i[b])

    def wait_idx(k, b):
        pltpu.make_async_copy(idx_hbm.at[w, k], idxv[b], sem_i[b]).wait()

    def issue_gather(b):
        pltpu.async_copy(x_hbm.at[idxv[b].at[0]], bufs[b], sem_g[b])

    def wait_gather(b):
        pltpu.make_async_copy(x_hbm.at[idxv[b].at[0]], bufs[b], sem_g[b]).wait()

    def scale(b):
        buf = bufs[b]

        def group_body(g, _):
            av = idxv[b][1, pl.ds(g * LANES, LANES)]
            for i in range(LANES):
                a = lax.bitcast_convert_type(av[i], jnp.float32)
                r = g * LANES + i
                for j in range(D // LANES):
                    buf[r, pl.ds(j * LANES, LANES)] = (
                        buf[r, pl.ds(j * LANES, LANES)] * a)
            return 0
        lax.fori_loop(0, CHUNK // LANES, group_body, 0)

    def scatter(b):
        pltpu.sync_copy(bufs[b], acc.at[idxv[b].at[2]], add=True)

    def zero_body(r, _):
        for j in range(D // LANES):
            buf0[r, pl.ds(j * LANES, LANES)] = jnp.zeros((LANES,), jnp.float32)
        return 0
    lax.fori_loop(0, ZCHUNK, zero_body, 0)
    for k in range(ROWS_PER_TILE // ZCHUNK):
        pltpu.sync_copy(buf0, acc.at[pl.ds(s * ROWS_PER_TILE + k * ZCHUNK, ZCHUNK)])
    plsc.subcore_barrier()

    pltpu.sync_copy(idx_hbm.at[w, 0], idx0)
    issue_gather(0)
    if nchunks > 1:
        issue_idx(1, 1)
    wait_gather(0)

    def process(k, b, nxt, nxt2):
        if nxt:
            wait_idx(k + 1, 1 - b)
            issue_gather(1 - b)
        scale(b)
        if nxt:
            wait_gather(1 - b)
        scatter(b)
        if nxt2:
            issue_idx(k + 2, b)

    def pair_body(p, _):
        process(2 * p, 0, True, True)
        process(2 * p + 1, 1, True, True)
        return 0
    m = max((nchunks - 2) // 2, 0)
    lax.fori_loop(0, m, pair_body, 0)
    for k in range(2 * m, nchunks):
        process(k, k % 2, k + 1 < nchunks, k + 2 < nchunks)

    plsc.subcore_barrier()
    for k in range(ROWS_PER_TILE // ZCHUNK):
        r0 = s * ROWS_PER_TILE + k * ZCHUNK
        pltpu.sync_copy(acc.at[pl.ds(r0, ZCHUNK)],
                        out_hbm.at[c, pl.ds(r0, ZCHUNK)])


@jax.jit
def _sc_spmm(x, idx):
    mesh = plsc.VectorSubcoreMesh(core_axis_name="c", subcore_axis_name="s")
    f = pl.kernel(
        _sc_spmm_body,
        out_type=jax.ShapeDtypeStruct((NC, N_PAD, D), jnp.float32),
        mesh=mesh,
        scratch_types=[
            pltpu.VMEM((3, CHUNK), jnp.int32),
            pltpu.VMEM((3, CHUNK), jnp.int32),
            pltpu.VMEM((CHUNK, D), jnp.float32),
            pltpu.VMEM((CHUNK, D), jnp.float32),
            pltpu.VMEM_SHARED((N_PAD, D), jnp.float32),
            pltpu.SemaphoreType.DMA,
            pltpu.SemaphoreType.DMA,
            pltpu.SemaphoreType.DMA,
            pltpu.SemaphoreType.DMA,
        ],
    )
    return f(x, idx)


def _tc_body(p_ref, ix_ref, w_ref, o_ref):
    hidden = (1.0 - ALPHA) * (p_ref[0] + p_ref[1]) + ALPHA * ix_ref[...]
    o_ref[...] = jnp.dot(hidden, w_ref[...], preferred_element_type=jnp.float32)


@jax.jit
def _tc_combine_matmul(partials, init_x, weight):
    blk = 1000
    grid = (N_NODES // blk,)
    return pl.pallas_call(
        _tc_body,
        grid=grid,
        in_specs=[
            pl.BlockSpec((NC, blk, D), lambda i: (0, i, 0)),
            pl.BlockSpec((blk, D), lambda i: (i, 0)),
            pl.BlockSpec((D, D), lambda i: (0, 0)),
        ],
        out_specs=pl.BlockSpec((blk, D), lambda i: (i, 0)),
        out_shape=jax.ShapeDtypeStruct((N_NODES, D), jnp.float32),
    )(partials, init_x, weight)


def kernel(x, edge_index, edge_attr, init_x, weight):
    e = edge_index.shape[1]
    nchunks = (e + NW * CHUNK - 1) // (NW * CHUNK)
    epw = nchunks * CHUNK
    ep = epw * NW
    pad = ep - e
    row = jnp.pad(jnp.asarray(edge_index[0], jnp.int32), (0, pad))
    col = jnp.pad(jnp.asarray(edge_index[1], jnp.int32), (0, pad))
    ea = jnp.pad(jnp.asarray(edge_attr, jnp.float32), (0, pad))
    ea_bits = lax.bitcast_convert_type(ea, jnp.int32)
    idx = jnp.stack(
        [col.reshape(NW, nchunks, CHUNK),
         ea_bits.reshape(NW, nchunks, CHUNK),
         row.reshape(NW, nchunks, CHUNK)], axis=2)
    partials = _sc_spmm(x, idx)
    return _tc_combine_matmul(partials, init_x, weight)


# speedup vs baseline: 6.7000x; 1.2463x over previous
import jax
import jax.numpy as jnp
from jax import lax
from jax.experimental import pallas as pl
from jax.experimental.pallas import tpu as pltpu
from jax.experimental.pallas import tpu_sc as plsc

N_NODES = 10000
D = 128
ALPHA = 0.1

NC = 2
NS = 16
NW = NC * NS
CHUNK = 128
LANES = 16

N_PAD = 10240
ROWS_PER_TILE = N_PAD // NS
ZCHUNK = 128


def _sc_spmm_body(x_hbm, idx_hbm, out_hbm,
                  idx0, idx1, row0, row1, buf0, buf1, acc,
                  sem_i0, sem_i1, sem_g0, sem_g1, sem_s0, sem_s1):
    c = lax.axis_index("c")
    s = lax.axis_index("s")
    w = c * NS + s
    nchunks = idx_hbm.shape[1]

    idxv = (idx0, idx1)
    rowv = (row0, row1)
    bufs = (buf0, buf1)
    sem_i = (sem_i0, sem_i1)
    sem_g = (sem_g0, sem_g1)
    sem_s = (sem_s0, sem_s1)

    def issue_idx(k, b):
        pltpu.async_copy(idx_hbm.at[w, k], idxv[b], sem_i[b])

    def wait_idx(k, b):
        pltpu.make_async_copy(idx_hbm.at[w, k], idxv[b], sem_i[b]).wait()

    def issue_gather(b):
        pltpu.async_copy(x_hbm.at[idxv[b].at[0]], bufs[b], sem_g[b])

    def wait_gather(b):
        pltpu.make_async_copy(x_hbm.at[idxv[b].at[0]], bufs[b], sem_g[b]).wait()

    def scale(b):
        buf = bufs[b]

        def group_body(g, _):
            av = idxv[b][1, pl.ds(g * LANES, LANES)]
            for i in range(LANES):
                a = lax.bitcast_convert_type(av[i], jnp.float32)
                r = g * LANES + i
                for j in range(D // LANES):
                    buf[r, pl.ds(j * LANES, LANES)] = (
                        buf[r, pl.ds(j * LANES, LANES)] * a)
            return 0
        lax.fori_loop(0, CHUNK // LANES, group_body, 0)

    def copy_rows(b):
        for g in range(CHUNK // LANES):
            rowv[b][pl.ds(g * LANES, LANES)] = idxv[b][2, pl.ds(g * LANES, LANES)]

    def issue_scatter(b):
        pltpu.async_copy(bufs[b], acc.at[rowv[b]], sem_s[b], add=True)

    def wait_scatter(b):
        pltpu.make_async_copy(bufs[b], acc.at[rowv[b]], sem_s[b]).wait()

    def zero_body(r, _):
        for j in range(D // LANES):
            buf0[r, pl.ds(j * LANES, LANES)] = jnp.zeros((LANES,), jnp.float32)
        return 0
    lax.fori_loop(0, ZCHUNK, zero_body, 0)
    for k in range(ROWS_PER_TILE // ZCHUNK):
        pltpu.sync_copy(buf0, acc.at[pl.ds(s * ROWS_PER_TILE + k * ZCHUNK, ZCHUNK)])
    plsc.subcore_barrier()

    pltpu.sync_copy(idx_hbm.at[w, 0], idx0)
    issue_gather(0)
    if nchunks > 1:
        issue_idx(1, 1)

    def process(k, b, first, nxt, nxt2):
        wait_gather(b)
        if not first:
            wait_scatter(1 - b)
        if nxt:
            wait_idx(k + 1, 1 - b)
            issue_gather(1 - b)
        scale(b)
        copy_rows(b)
        issue_scatter(b)
        if nxt2:
            issue_idx(k + 2, b)

    def pair_body(p, _):
        process(2 * p + 2, 0, False, True, True)
        process(2 * p + 3, 1, False, True, True)
        return 0
    if nchunks <= 5:
        for k in range(nchunks):
            process(k, k % 2, k == 0, k + 1 < nchunks, k + 2 < nchunks)
    else:
        process(0, 0, True, True, True)
        process(1, 1, False, True, True)
        m = (nchunks - 2 - 3) // 2
        lax.fori_loop(0, m, pair_body, 0)
        for k in range(2 + 2 * m, nchunks):
            process(k, k % 2, False, k + 1 < nchunks, k + 2 < nchunks)
    wait_scatter((nchunks - 1) % 2)

    plsc.subcore_barrier()
    for k in range(ROWS_PER_TILE // ZCHUNK):
        r0 = s * ROWS_PER_TILE + k * ZCHUNK
        pltpu.sync_copy(acc.at[pl.ds(r0, ZCHUNK)],
                        out_hbm.at[c, pl.ds(r0, ZCHUNK)])


@jax.jit
def _sc_spmm(x, idx):
    mesh = plsc.VectorSubcoreMesh(core_axis_name="c", subcore_axis_name="s")
    f = pl.kernel(
        _sc_spmm_body,
        out_type=jax.ShapeDtypeStruct((NC, N_PAD, D), jnp.float32),
        mesh=mesh,
        scratch_types=[
            pltpu.VMEM((3, CHUNK), jnp.int32),
            pltpu.VMEM((3, CHUNK), jnp.int32),
            pltpu.VMEM((CHUNK,), jnp.int32),
            pltpu.VMEM((CHUNK,), jnp.int32),
            pltpu.VMEM((CHUNK, D), jnp.float32),
            pltpu.VMEM((CHUNK, D), jnp.float32),
            pltpu.VMEM_SHARED((N_PAD, D), jnp.float32),
            pltpu.SemaphoreType.DMA,
            pltpu.SemaphoreType.DMA,
            pltpu.SemaphoreType.DMA,
            pltpu.SemaphoreType.DMA,
            pltpu.SemaphoreType.DMA,
            pltpu.SemaphoreType.DMA,
        ],
    )
    return f(x, idx)


def _tc_body(p_ref, ix_ref, w_ref, o_ref):
    hidden = (1.0 - ALPHA) * (p_ref[0] + p_ref[1]) + ALPHA * ix_ref[...]
    o_ref[...] = jnp.dot(hidden, w_ref[...], preferred_element_type=jnp.float32)


@jax.jit
def _tc_combine_matmul(partials, init_x, weight):
    blk = 1000
    grid = (N_NODES // blk,)
    return pl.pallas_call(
        _tc_body,
        grid=grid,
        in_specs=[
            pl.BlockSpec((NC, blk, D), lambda i: (0, i, 0)),
            pl.BlockSpec((blk, D), lambda i: (i, 0)),
            pl.BlockSpec((D, D), lambda i: (0, 0)),
        ],
        out_specs=pl.BlockSpec((blk, D), lambda i: (i, 0)),
        out_shape=jax.ShapeDtypeStruct((N_NODES, D), jnp.float32),
    )(partials, init_x, weight)


def kernel(x, edge_index, edge_attr, init_x, weight):
    e = edge_index.shape[1]
    nchunks = (e + NW * CHUNK - 1) // (NW * CHUNK)
    epw = nchunks * CHUNK
    ep = epw * NW
    pad = ep - e
    row = jnp.pad(jnp.asarray(edge_index[0], jnp.int32), (0, pad))
    col = jnp.pad(jnp.asarray(edge_index[1], jnp.int32), (0, pad))
    ea = jnp.pad(jnp.asarray(edge_attr, jnp.float32), (0, pad))
    ea_bits = lax.bitcast_convert_type(ea, jnp.int32)
    idx = jnp.stack(
        [col.reshape(NW, nchunks, CHUNK),
         ea_bits.reshape(NW, nchunks, CHUNK),
         row.reshape(NW, nchunks, CHUNK)], axis=2)
    partials = _sc_spmm(x, idx)
    return _tc_combine_matmul(partials, init_x, weight)
